# Initial kernel scaffold; baseline (speedup 1.0000x reference)
import jax
import jax.numpy as jnp
from jax.experimental import pallas as pl


def kernel(x, edge_index, edge_attr):
    raise NotImplementedError("write your pallas kernel here")


# speedup vs baseline: 3.2379x; 3.2379x over previous
import functools

import jax
import jax.numpy as jnp
from jax import lax
from jax.experimental import pallas as pl
from jax.experimental.pallas import tpu as pltpu
from jax.experimental.pallas import tpu_sc as plsc

N_NODES = 10000
N_EDGES = 320000
FDIM = 16
W = 128
NC = 2
NS = 16
EDGES_PER_CORE = N_EDGES // NC
EDGES_PER_TILE = EDGES_PER_CORE // NS
CH = 80
N_CHUNKS = EDGES_PER_TILE // CH
N_PAD = 10240
ROWS_PER_TILE = N_PAD // NS
ZROWS = 128


def _sc_scatter_partials(dst, edge_attr):
    mesh = plsc.VectorSubcoreMesh(core_axis_name="c", subcore_axis_name="s")
    out_type = jax.ShapeDtypeStruct((NC, N_PAD, W), jnp.float32)

    @functools.partial(
        pl.kernel,
        out_type=out_type,
        mesh=mesh,
        scratch_types=[
            pltpu.VMEM_SHARED((N_PAD, W), jnp.float32),
            pltpu.VMEM((CH,), jnp.int32),
            pltpu.VMEM((CH, FDIM), jnp.float32),
            pltpu.VMEM((CH, W), jnp.float32),
            pltpu.VMEM((ZROWS, W), jnp.float32),
        ],
    )
    def scatter_kernel(dst_hbm, attr_hbm, pacc_hbm,
                       acc_sh, idx_v, attr_v, stage_v, zbuf):
        c = lax.axis_index("c")
        s = lax.axis_index("s")

        zv = jnp.zeros((16,), jnp.float32)

        @pl.loop(0, ZROWS)
        def _(i):
            @pl.loop(0, W // 16)
            def _(j):
                zbuf[i, pl.ds(j * 16, 16)] = zv

        lane = lax.iota(jnp.int32, 16)
        onehot = jnp.where(lane == 0, 1.0, 0.0).astype(jnp.float32)

        @pl.loop(0, CH)
        def _(e):
            stage_v[e, pl.ds(16, 16)] = onehot

            @pl.loop(2, W // 16)
            def _(j):
                stage_v[e, pl.ds(j * 16, 16)] = zv

        row0 = s * ROWS_PER_TILE

        @pl.loop(0, ROWS_PER_TILE // ZROWS)
        def _(b):
            pltpu.sync_copy(zbuf, acc_sh.at[pl.ds(row0 + b * ZROWS, ZROWS)])

        plsc.subcore_barrier()

        wid = c * NS + s
        base = wid * EDGES_PER_TILE

        @pl.loop(0, N_CHUNKS)
        def _(j):
            off = base + j * CH
            pltpu.sync_copy(dst_hbm.at[pl.ds(off, CH)], idx_v)
            pltpu.sync_copy(attr_hbm.at[pl.ds(off, CH)], attr_v)

            @pl.loop(0, CH)
            def _(e):
                stage_v[e, pl.ds(0, 16)] = attr_v[e, :]

            pltpu.sync_copy(stage_v, acc_sh.at[idx_v], add=True)

        plsc.subcore_barrier()
        pltpu.sync_copy(acc_sh.at[pl.ds(row0, ROWS_PER_TILE)],
                        pacc_hbm.at[c, pl.ds(row0, ROWS_PER_TILE)])

    return scatter_kernel(dst, edge_attr)


def _divide_body(pa_ref, o_ref):
    s = pa_ref[0] + pa_ref[1]
    cnt = jnp.maximum(s[:, 16:17], 1.0)
    o_ref[...] = s / cnt


def _tc_combine_divide(pacc):
    out = pl.pallas_call(
        _divide_body,
        out_shape=jax.ShapeDtypeStruct((N_PAD, W), jnp.float32),
    )(pacc)
    return out[:N_NODES, :FDIM]


def kernel(x, edge_index, edge_attr):
    del x
    dst = edge_index[1]
    pacc = _sc_scatter_partials(dst, edge_attr)
    return _tc_combine_divide(pacc)


# speedup vs baseline: 5.5580x; 1.7166x over previous
import functools

import jax
import jax.numpy as jnp
from jax import lax
from jax.experimental import pallas as pl
from jax.experimental.pallas import tpu as pltpu
from jax.experimental.pallas import tpu_sc as plsc

N_NODES = 10000
N_EDGES = 320000
FDIM = 16
W = 128
NC = 2
NS = 16
EDGES_PER_CORE = N_EDGES // NC
EDGES_PER_TILE = EDGES_PER_CORE // NS
CH = 80
N_CHUNKS = EDGES_PER_TILE // CH
NBUF = 2
N_OUTER = N_CHUNKS // NBUF
N_TAIL = N_CHUNKS - N_OUTER * NBUF
N_PAD = 10240
ROWS_PER_TILE = N_PAD // NS


def _sc_scatter_partials(dst, edge_attr):
    mesh = plsc.VectorSubcoreMesh(core_axis_name="c", subcore_axis_name="s")
    out_type = jax.ShapeDtypeStruct((NC, N_PAD, W), jnp.float32)

    scratch = (
        [pltpu.VMEM_SHARED((N_PAD, W), jnp.float32)]
        + [pltpu.VMEM((CH,), jnp.int32) for _ in range(NBUF)]
        + [pltpu.VMEM((CH, FDIM), jnp.float32) for _ in range(NBUF)]
        + [pltpu.VMEM((CH,), jnp.int32) for _ in range(NBUF)]
        + [pltpu.VMEM((CH, W), jnp.float32) for _ in range(NBUF)]
        + [pltpu.SemaphoreType.DMA for _ in range(2 * NBUF)]
    )

    @functools.partial(pl.kernel, out_type=out_type, mesh=mesh,
                       scratch_types=scratch)
    def scatter_kernel(dst_hbm, attr_hbm, pacc_hbm, acc_sh, *bufs):
        idx_v = bufs[0:NBUF]
        attr_v = bufs[NBUF:2 * NBUF]
        sidx_v = bufs[2 * NBUF:3 * NBUF]
        stage_v = bufs[3 * NBUF:4 * NBUF]
        sem_load = bufs[4 * NBUF:4 * NBUF + NBUF]
        sem_scat = bufs[4 * NBUF + NBUF:4 * NBUF + 2 * NBUF]

        c = lax.axis_index("c")
        s = lax.axis_index("s")

        zv = jnp.zeros((16,), jnp.float32)

        @pl.loop(0, CH)
        def _(e):
            @pl.loop(0, W // 16)
            def _(j):
                stage_v[0][e, pl.ds(j * 16, 16)] = zv

        row0 = s * ROWS_PER_TILE

        @pl.loop(0, ROWS_PER_TILE // CH)
        def _(b):
            pltpu.sync_copy(stage_v[0], acc_sh.at[pl.ds(row0 + b * CH, CH)])

        lane = lax.iota(jnp.int32, 16)
        onehot = jnp.where(lane == 0, 1.0, 0.0).astype(jnp.float32)
        for b in range(NBUF):
            @pl.loop(0, CH)
            def _(e, b=b):
                stage_v[b][e, pl.ds(16, 16)] = onehot
                if b != 0:
                    @pl.loop(2, W // 16)
                    def _(j):
                        stage_v[b][e, pl.ds(j * 16, 16)] = zv

        plsc.subcore_barrier()

        wid = c * NS + s
        base = wid * EDGES_PER_TILE

        def fire_loads(k, b):
            off = base + k * CH
            pltpu.async_copy(dst_hbm.at[pl.ds(off, CH)], idx_v[b], sem_load[b])
            pltpu.async_copy(attr_hbm.at[pl.ds(off, CH)], attr_v[b], sem_load[b])

        def wait_loads(b):
            pltpu.make_async_copy(dst_hbm.at[pl.ds(0, CH)], idx_v[b],
                                  sem_load[b]).wait()
            pltpu.make_async_copy(attr_hbm.at[pl.ds(0, CH)], attr_v[b],
                                  sem_load[b]).wait()

        def wait_scatter(b):
            pltpu.make_async_copy(stage_v[b], acc_sh.at[sidx_v[b]],
                                  sem_scat[b]).wait()

        for b in range(NBUF):
            fire_loads(b, b)

        @pl.loop(0, N_OUTER)
        def _(t):
            j0 = t * NBUF
            for b in range(NBUF):
                wait_loads(b)

                @pl.when(t >= 1)
                def _(b=b):
                    wait_scatter(b)

                for u in range(CH // 16):
                    sidx_v[b][pl.ds(u * 16, 16)] = idx_v[b][pl.ds(u * 16, 16)]

                @pl.loop(0, CH)
                def _(e, b=b):
                    stage_v[b][e, pl.ds(0, 16)] = attr_v[b][e, :]

                @pl.when(t < N_OUTER - 1)
                def _(b=b):
                    fire_loads(j0 + b + NBUF, b)

                pltpu.async_copy(stage_v[b], acc_sh.at[sidx_v[b]],
                                 sem_scat[b], add=True)

        for b in range(NBUF):
            wait_scatter(b)

        for r in range(N_TAIL):
            k = N_OUTER * NBUF + r
            off = base + k * CH
            pltpu.sync_copy(dst_hbm.at[pl.ds(off, CH)], idx_v[0])
            pltpu.sync_copy(attr_hbm.at[pl.ds(off, CH)], attr_v[0])

            @pl.loop(0, CH)
            def _(e):
                stage_v[0][e, pl.ds(0, 16)] = attr_v[0][e, :]

            pltpu.sync_copy(stage_v[0], acc_sh.at[idx_v[0]], add=True)

        plsc.subcore_barrier()
        pltpu.sync_copy(acc_sh.at[pl.ds(row0, ROWS_PER_TILE)],
                        pacc_hbm.at[c, pl.ds(row0, ROWS_PER_TILE)])

    return scatter_kernel(dst, edge_attr)


def _divide_body(pa_ref, o_ref):
    s = pa_ref[0] + pa_ref[1]
    cnt = jnp.maximum(s[:, 16:17], 1.0)
    o_ref[...] = s / cnt


def _tc_combine_divide(pacc):
    out = pl.pallas_call(
        _divide_body,
        out_shape=jax.ShapeDtypeStruct((N_PAD, W), jnp.float32),
    )(pacc)
    return out[:N_NODES, :FDIM]


def kernel(x, edge_index, edge_attr):
    del x
    dst = edge_index[1]
    pacc = _sc_scatter_partials(dst, edge_attr)
    return _tc_combine_divide(pacc)


# speedup vs baseline: 6.1777x; 1.1115x over previous
import functools

import jax
import jax.numpy as jnp
from jax import lax
from jax.experimental import pallas as pl
from jax.experimental.pallas import tpu as pltpu
from jax.experimental.pallas import tpu_sc as plsc

N_NODES = 10000
N_EDGES = 320000
FDIM = 16
W = 128
NC = 2
NS = 16
EDGES_PER_CORE = N_EDGES // NC
EDGES_PER_TILE = EDGES_PER_CORE // NS
CH = 80
CHS = 40
NL = 3
NSC = 2
N_CHUNKS = EDGES_PER_TILE // CH
N_PAD = 10240
ROWS_PER_TILE = N_PAD // NS


def _sc_scatter_partials(dst, edge_attr):
    mesh = plsc.VectorSubcoreMesh(core_axis_name="c", subcore_axis_name="s")
    out_type = jax.ShapeDtypeStruct((NC, N_PAD, W), jnp.float32)

    scratch = [
        pltpu.VMEM_SHARED((N_PAD, W), jnp.float32),
        pltpu.VMEM((NL * CH,), jnp.int32),
        pltpu.VMEM((NL * CH, FDIM), jnp.float32),
        pltpu.VMEM((CHS,), jnp.int32),
        pltpu.VMEM((CHS,), jnp.int32),
        pltpu.VMEM((CHS, W), jnp.float32),
        pltpu.VMEM((CHS, W), jnp.float32),
        pltpu.SemaphoreType.DMA((NL,)),
        pltpu.SemaphoreType.DMA,
        pltpu.SemaphoreType.DMA,
    ]

    @functools.partial(pl.kernel, out_type=out_type, mesh=mesh,
                       scratch_types=scratch)
    def scatter_kernel(dst_hbm, attr_hbm, pacc_hbm, acc_sh,
                       idx_all, attr_all, sidx0, sidx1, stage0, stage1,
                       sem_load, sem_s0, sem_s1):
        sidx_v = (sidx0, sidx1)
        stage_v = (stage0, stage1)
        sem_scat = (sem_s0, sem_s1)

        c = lax.axis_index("c")
        s = lax.axis_index("s")

        zv = jnp.zeros((16,), jnp.float32)

        @pl.loop(0, CHS)
        def _(e):
            @pl.loop(0, W // 16)
            def _(j):
                stage_v[0][e, pl.ds(j * 16, 16)] = zv

        row0 = s * ROWS_PER_TILE

        @pl.loop(0, ROWS_PER_TILE // CHS)
        def _(b):
            pltpu.sync_copy(stage_v[0], acc_sh.at[pl.ds(row0 + b * CHS, CHS)])

        lane = lax.iota(jnp.int32, 16)
        onehot = jnp.where(lane == 0, 1.0, 0.0).astype(jnp.float32)
        for b in range(NSC):
            @pl.loop(0, CHS)
            def _(e, b=b):
                stage_v[b][e, pl.ds(16, 16)] = onehot
                if b != 0:
                    @pl.loop(2, W // 16)
                    def _(j):
                        stage_v[b][e, pl.ds(j * 16, 16)] = zv

        plsc.subcore_barrier()

        wid = c * NS + s
        base = wid * EDGES_PER_TILE

        def fire_loads(k, slot):
            off = base + k * CH
            loc = slot * CH
            pltpu.async_copy(dst_hbm.at[pl.ds(off, CH)],
                             idx_all.at[pl.ds(loc, CH)], sem_load.at[slot])
            pltpu.async_copy(attr_hbm.at[pl.ds(off, CH)],
                             attr_all.at[pl.ds(loc, CH)], sem_load.at[slot])

        def wait_loads(slot):
            loc = slot * CH
            pltpu.make_async_copy(dst_hbm.at[pl.ds(0, CH)],
                                  idx_all.at[pl.ds(loc, CH)],
                                  sem_load.at[slot]).wait()
            pltpu.make_async_copy(attr_hbm.at[pl.ds(0, CH)],
                                  attr_all.at[pl.ds(loc, CH)],
                                  sem_load.at[slot]).wait()

        def wait_scatter(h):
            pltpu.make_async_copy(stage_v[h], acc_sh.at[sidx_v[h]],
                                  sem_scat[h]).wait()

        @pl.loop(0, NL)
        def _(b):
            fire_loads(b, b)

        @pl.loop(0, N_CHUNKS)
        def _(k):
            slot = lax.rem(k, NL)
            loc = slot * CH
            wait_loads(slot)
            for h in range(NSC):
                @pl.when(k > 0)
                def _(h=h):
                    wait_scatter(h)

                e0 = loc + h * CHS
                sidx_v[h][pl.ds(0, 16)] = idx_all[pl.ds(e0, 16)]
                sidx_v[h][pl.ds(16, 16)] = idx_all[pl.ds(e0 + 16, 16)]
                sidx_v[h][pl.ds(CHS - 16, 16)] = idx_all[pl.ds(e0 + CHS - 16, 16)]

                @pl.loop(0, CHS)
                def _(r, h=h, e0=e0):
                    stage_v[h][r, pl.ds(0, 16)] = attr_all[e0 + r, :]

                pltpu.async_copy(stage_v[h], acc_sh.at[sidx_v[h]],
                                 sem_scat[h], add=True)

            @pl.when(k + NL < N_CHUNKS)
            def _():
                fire_loads(k + NL, slot)

        for h in range(NSC):
            wait_scatter(h)

        plsc.subcore_barrier()
        pltpu.sync_copy(acc_sh.at[pl.ds(row0, ROWS_PER_TILE)],
                        pacc_hbm.at[c, pl.ds(row0, ROWS_PER_TILE)])

    return scatter_kernel(dst, edge_attr)


def _divide_body(pa_ref, o_ref):
    s = pa_ref[0] + pa_ref[1]
    cnt = jnp.maximum(s[:, 16:17], 1.0)
    o_ref[...] = s / cnt


def _tc_combine_divide(pacc):
    out = pl.pallas_call(
        _divide_body,
        out_shape=jax.ShapeDtypeStruct((N_PAD, W), jnp.float32),
    )(pacc)
    return out[:N_NODES, :FDIM]


def kernel(x, edge_index, edge_attr):
    del x
    dst = edge_index[1]
    pacc = _sc_scatter_partials(dst, edge_attr)
    return _tc_combine_divide(pacc)
